# baseline (speedup 1.0000x reference)
import functools

import jax
import jax.numpy as jnp
from jax import lax
from jax.experimental import pallas as pl
from jax.experimental.pallas import tpu as pltpu
from jax.experimental.pallas import tpu_sc as plsc

B = 16384
D = 32
NC = 2
NS = 16
BPS = B // NS
CHUNK = 128
NCHUNK = BPS // CHUNK
VP_DIV = 1001


def _make_sc_kernel(vps):
    mesh = plsc.VectorSubcoreMesh(core_axis_name="c", subcore_axis_name="s")

    @functools.partial(
        pl.kernel,
        mesh=mesh,
        out_type=jax.ShapeDtypeStruct((5 * D, B), jnp.float32),
        compiler_params=pltpu.CompilerParams(
            use_tc_tiling_on_sc=False, needs_layout_passes=False
        ),
        scratch_types=[
            pltpu.VMEM((5, BPS), jnp.int32),
            pltpu.VMEM((16, CHUNK), jnp.int32),
            pltpu.VMEM((2, 16, BPS), jnp.float32),
            pltpu.VMEM((16, VP_DIV), jnp.float32),
            pltpu.SemaphoreType.DMA,
            pltpu.SemaphoreType.DMA,
            pltpu.SemaphoreType.DMA,
            pltpu.SemaphoreType.DMA,
            pltpu.SemaphoreType.DMA,
        ],
    )
    def k(i0, i1, i2, i3, i4, wdiv_t, f1, f2, f3, f4, out,
          ids_v, idxd_v, dst_v, div_v, sem_i, sem_g, sem_w0, sem_w1, sem_st):
        c = lax.axis_index("c")
        s = lax.axis_index("s")
        base = s * BPS
        sem_w = (sem_w0, sem_w1)

        id_copies = [
            pltpu.async_copy(ih.at[pl.ds(base, BPS)], ids_v.at[l], sem_i)
            for l, ih in enumerate((i0, i1, i2, i3, i4))
        ]
        st_copy = pltpu.async_copy(
            wdiv_t.at[pl.ds(c * 16, 16), :], div_v, sem_st
        )
        for cp in id_copies:
            cp.wait()

        write_copies = []

        def flush(li, lvl_row):
            wc = pltpu.async_copy(
                dst_v.at[li % 2],
                out.at[pl.ds(lvl_row, 16), pl.ds(base, BPS)],
                sem_w[li % 2],
            )
            write_copies.append(wc)

        for li, (lvl, tab) in enumerate(
            ((1, f1), (2, f2), (3, f3), (4, f4))
        ):
            vp = vps[li]
            if li >= 2:
                write_copies[li - 2].wait()
            offs = [(c * 16 + d) * vp for d in range(16)]

            def chunk_body(ch, _, lvl=lvl, offs=offs, tab=tab, li=li):
                cb = ch * CHUNK
                for g in range(CHUNK // 16):
                    iv = ids_v[lvl, pl.ds(cb + g * 16, 16)]
                    for d in range(16):
                        idxd_v[d, pl.ds(g * 16, 16)] = iv + offs[d]
                gs = []
                for d in range(16):
                    gs.append(
                        pltpu.async_copy(
                            tab.at[idxd_v.at[d]],
                            dst_v.at[li % 2, d, pl.ds(cb, CHUNK)],
                            sem_g,
                        )
                    )
                for g_ in gs:
                    g_.wait()
                return 0

            lax.fori_loop(0, NCHUNK, chunk_body, 0, unroll=False)
            flush(li, 32 * lvl + 16 * c)

        write_copies[2].wait()
        st_copy.wait()
        dsel = [jnp.full((16,), d, dtype=jnp.int32) for d in range(16)]

        def div_body(g, _):
            iv = ids_v[0, pl.ds(g * 16, 16)]
            for d in range(16):
                vals = plsc.load_gather(div_v, [dsel[d], iv])
                dst_v[0, d, pl.ds(g * 16, 16)] = vals
            return 0

        lax.fori_loop(0, BPS // 16, div_body, 0, unroll=False)
        flush(4, 16 * c)

        write_copies[3].wait()
        write_copies[4].wait()

    return k


_SC_EMBED_CACHE = {}


def _get_sc_embed(vps):
    if vps not in _SC_EMBED_CACHE:
        _SC_EMBED_CACHE[vps] = _make_sc_kernel(vps)
    return _SC_EMBED_CACHE[vps]


def kernel(division_ids, department_ids, class_ids, subclass_ids, group_ids,
           W_division, W_department, W_class, W_subclass, W_group):
    flats = [
        w.T.reshape(-1)
        for w in (W_department, W_class, W_subclass, W_group)
    ]
    vps = tuple(
        w.shape[0] for w in (W_department, W_class, W_subclass, W_group)
    )
    fn = _get_sc_embed(vps)
    out_t = fn(division_ids, department_ids, class_ids, subclass_ids,
               group_ids, W_division.T, *flats)
    return out_t.T


# speedup vs baseline: 1.0885x; 1.0885x over previous
import functools

import jax
import jax.numpy as jnp
from jax import lax
from jax.experimental import pallas as pl
from jax.experimental.pallas import tpu as pltpu
from jax.experimental.pallas import tpu_sc as plsc

B = 16384
D = 32
NC = 2
NS = 16
BPS = B // NS
CHUNK = 128
NCHUNK = BPS // CHUNK
VP_DIV = 1001


def _make_sc_kernel(vps):
    mesh = plsc.VectorSubcoreMesh(core_axis_name="c", subcore_axis_name="s")

    @functools.partial(
        pl.kernel,
        mesh=mesh,
        out_type=jax.ShapeDtypeStruct((5 * D, B), jnp.float32),
        compiler_params=pltpu.CompilerParams(
            use_tc_tiling_on_sc=False, needs_layout_passes=False
        ),
        scratch_types=[
            pltpu.VMEM((5, BPS), jnp.int32),
            pltpu.VMEM((16, CHUNK), jnp.int32),
            pltpu.VMEM((2, 16, BPS), jnp.float32),
            pltpu.VMEM((16, VP_DIV), jnp.float32),
            pltpu.SemaphoreType.DMA,
            pltpu.SemaphoreType.DMA,
            pltpu.SemaphoreType.DMA,
            pltpu.SemaphoreType.DMA,
            pltpu.SemaphoreType.DMA,
        ],
    )
    def k(i0, i1, i2, i3, i4, wdiv_t, f1, f2, f3, f4, out,
          ids_v, idxd_v, dst_v, div_v, sem_i, sem_g, sem_w0, sem_w1, sem_st):
        c = lax.axis_index("c")
        s = lax.axis_index("s")
        base = s * BPS
        sem_w = (sem_w0, sem_w1)

        id_copies = [
            pltpu.async_copy(ih.at[pl.ds(base, BPS)], ids_v.at[l], sem_i)
            for l, ih in enumerate((i0, i1, i2, i3, i4))
        ]
        st_copy = pltpu.async_copy(
            wdiv_t.at[pl.ds(c * 16, 16), :], div_v, sem_st
        )
        for cp in id_copies:
            cp.wait()

        write_copies = []

        def flush(li, lvl_row):
            wc = pltpu.async_copy(
                dst_v.at[li % 2],
                out.at[pl.ds(lvl_row, 16), pl.ds(base, BPS)],
                sem_w[li % 2],
            )
            write_copies.append(wc)

        for li, (lvl, tab) in enumerate(
            ((1, f1), (2, f2), (3, f3), (4, f4))
        ):
            vp = vps[li]
            if li >= 2:
                write_copies[li - 2].wait()
            offs = [(c * 16 + d) * vp for d in range(16)]

            def chunk_body(ch, _, lvl=lvl, offs=offs, tab=tab, li=li):
                cb = ch * CHUNK
                for g in range(CHUNK // 16):
                    iv = ids_v[lvl, pl.ds(cb + g * 16, 16)]
                    for d in range(16):
                        idxd_v[d, pl.ds(g * 16, 16)] = iv + offs[d]
                gs = []
                for d in range(16):
                    gs.append(
                        pltpu.async_copy(
                            tab.at[idxd_v.at[d]],
                            dst_v.at[li % 2, d, pl.ds(cb, CHUNK)],
                            sem_g,
                        )
                    )
                for g_ in gs:
                    g_.wait()
                return 0

            lax.fori_loop(0, NCHUNK, chunk_body, 0, unroll=False)
            flush(li, 32 * lvl + 16 * c)

        write_copies[2].wait()
        st_copy.wait()
        dsel = [jnp.full((16,), d, dtype=jnp.int32) for d in range(16)]

        def div_body(g, _):
            iv = ids_v[0, pl.ds(g * 16, 16)]
            for d in range(16):
                vals = plsc.load_gather(div_v, [dsel[d], iv])
                dst_v[0, d, pl.ds(g * 16, 16)] = vals
            return 0

        lax.fori_loop(0, BPS // 16, div_body, 0, unroll=False)
        flush(4, 16 * c)

        write_copies[3].wait()
        write_copies[4].wait()

    return k


_SC_EMBED_CACHE = {}


def _get_sc_embed(vps):
    if vps not in _SC_EMBED_CACHE:
        _SC_EMBED_CACHE[vps] = _make_sc_kernel(vps)
    return _SC_EMBED_CACHE[vps]


def kernel(division_ids, department_ids, class_ids, subclass_ids, group_ids,
           W_division, W_department, W_class, W_subclass, W_group):
    def flat(w):
        vp = w.shape[0]
        vpad = -(-vp // 128) * 128
        wp = jnp.pad(w, ((0, vpad - vp), (0, 0)))
        return wp.T.reshape(-1), vpad

    pairs = [flat(w) for w in (W_department, W_class, W_subclass, W_group)]
    flats = [p[0] for p in pairs]
    vps = tuple(p[1] for p in pairs)
    fn = _get_sc_embed(vps)
    out_t = fn(division_ids, department_ids, class_ids, subclass_ids,
               group_ids, W_division.T, *flats)
    return out_t.T
